# Initial kernel scaffold; baseline (speedup 1.0000x reference)
import jax
import jax.numpy as jnp
from jax.experimental import pallas as pl


def kernel(x, edge_i, edge_j, edge_f, W1, b1, W2, b2, Wl, bl):
    raise NotImplementedError("write your pallas kernel here")


# speedup vs baseline: 2.5738x; 2.5738x over previous
import functools

import jax
import jax.numpy as jnp
from jax import lax
from jax.experimental import pallas as pl
from jax.experimental.pallas import tpu as pltpu
from jax.experimental.pallas import tpu_sc as plsc

N_NODES = 10000
N_EDGES = 320000
ATOM_DIM = 128
EDGE_DIM = 16
HIDDEN = 128

NC = 2
NS = 16
NW = NC * NS
EPW = N_EDGES // NW
CB = 80
CHUNKS = EPW // CB


def _sc_gather(x, edge_j):
    mesh = plsc.VectorSubcoreMesh(core_axis_name="c", subcore_axis_name="s")

    @functools.partial(
        pl.kernel, mesh=mesh,
        out_type=jax.ShapeDtypeStruct((N_EDGES, ATOM_DIM), jnp.float32),
        scratch_types=[
            pltpu.VMEM((CB,), jnp.int32),
            pltpu.VMEM((CB, ATOM_DIM), jnp.float32),
            pltpu.SemaphoreType.DMA,
        ],
    )
    def k(x_hbm, ej_hbm, out_hbm, idx_v, rows_v, sem):
        wid = lax.axis_index("s") * NC + lax.axis_index("c")
        base = wid * EPW

        def body(j, _):
            off = base + j * CB
            pltpu.sync_copy(ej_hbm.at[pl.ds(off, CB)], idx_v)
            pltpu.async_copy(x_hbm.at[idx_v], rows_v, sem).wait()
            pltpu.sync_copy(rows_v, out_hbm.at[pl.ds(off, CB)])
            return 0

        lax.fori_loop(0, CHUNKS, body, 0)

    return k(x, edge_j)


AGG_ROWS = 10240


def _sc_scatter_add(m, edge_i, zeros_nh):
    mesh = plsc.VectorSubcoreMesh(core_axis_name="c", subcore_axis_name="s")
    rows_per_sub = AGG_ROWS // NS

    @functools.partial(
        pl.kernel, mesh=mesh,
        out_type=jax.ShapeDtypeStruct((NC, AGG_ROWS, HIDDEN), jnp.float32),
        scratch_types=[
            pltpu.VMEM((CB,), jnp.int32),
            pltpu.VMEM((CB, HIDDEN), jnp.float32),
            pltpu.VMEM_SHARED((AGG_ROWS, HIDDEN), jnp.float32),
            pltpu.SemaphoreType.DMA,
        ],
    )
    def k(m_hbm, ei_hbm, z_hbm, out_hbm, idx_v, rows_v, agg_sh, sem):
        cid = lax.axis_index("c")
        sid = lax.axis_index("s")
        wid = sid * NC + cid
        pltpu.sync_copy(z_hbm.at[pl.ds(sid * rows_per_sub, rows_per_sub)],
                        agg_sh.at[pl.ds(sid * rows_per_sub, rows_per_sub)])
        plsc.subcore_barrier()

        base = wid * EPW

        def body(j, _):
            off = base + j * CB
            pltpu.sync_copy(ei_hbm.at[pl.ds(off, CB)], idx_v)
            pltpu.sync_copy(m_hbm.at[pl.ds(off, CB)], rows_v)
            pltpu.sync_copy(rows_v, agg_sh.at[idx_v], add=True)
            return 0

        lax.fori_loop(0, CHUNKS, body, 0)
        plsc.subcore_barrier()
        pltpu.sync_copy(agg_sh.at[pl.ds(sid * rows_per_sub, rows_per_sub)],
                        out_hbm.at[cid, pl.ds(sid * rows_per_sub, rows_per_sub)])

    return k(m, edge_i, zeros_nh)


def _tc_edge_mlp(x_j, edge_f, W1a, W1b, b1, W2, b2):
    BE = 2560

    def body(xj_ref, f_ref, w1a, w1b, b1r, w2, b2r, out_ref):
        h = jnp.dot(xj_ref[...], w1a[...], preferred_element_type=jnp.float32)
        h = h + jnp.dot(f_ref[...], w1b[...], preferred_element_type=jnp.float32)
        h = h + b1r[...]
        h = h * jax.nn.sigmoid(h)
        mm = jnp.dot(h, w2[...], preferred_element_type=jnp.float32) + b2r[...]
        out_ref[...] = mm * jax.nn.sigmoid(mm)

    return pl.pallas_call(
        body,
        grid=(N_EDGES // BE,),
        in_specs=[
            pl.BlockSpec((BE, ATOM_DIM), lambda i: (i, 0)),
            pl.BlockSpec((BE, EDGE_DIM), lambda i: (i, 0)),
            pl.BlockSpec((ATOM_DIM, HIDDEN), lambda i: (0, 0)),
            pl.BlockSpec((EDGE_DIM, HIDDEN), lambda i: (0, 0)),
            pl.BlockSpec((1, HIDDEN), lambda i: (0, 0)),
            pl.BlockSpec((HIDDEN, HIDDEN), lambda i: (0, 0)),
            pl.BlockSpec((1, HIDDEN), lambda i: (0, 0)),
        ],
        out_specs=pl.BlockSpec((BE, HIDDEN), lambda i: (i, 0)),
        out_shape=jax.ShapeDtypeStruct((N_EDGES, HIDDEN), jnp.float32),
    )(x_j, edge_f, W1a, W1b, b1.reshape(1, HIDDEN), W2, b2.reshape(1, HIDDEN))


def _tc_final(x, a0, a1, Wl, bl):
    BN = 1000

    def body(x_ref, a0_ref, a1_ref, wl, blr, out_ref):
        a = a0_ref[...] + a1_ref[...]
        t = x_ref[...] + jnp.dot(a, wl[...], preferred_element_type=jnp.float32)
        t = t + blr[...]
        out_ref[...] = t * jax.nn.sigmoid(t)

    return pl.pallas_call(
        body,
        grid=(N_NODES // BN,),
        in_specs=[
            pl.BlockSpec((BN, ATOM_DIM), lambda i: (i, 0)),
            pl.BlockSpec((BN, HIDDEN), lambda i: (i, 0)),
            pl.BlockSpec((BN, HIDDEN), lambda i: (i, 0)),
            pl.BlockSpec((HIDDEN, ATOM_DIM), lambda i: (0, 0)),
            pl.BlockSpec((1, ATOM_DIM), lambda i: (0, 0)),
        ],
        out_specs=pl.BlockSpec((BN, ATOM_DIM), lambda i: (i, 0)),
        out_shape=jax.ShapeDtypeStruct((N_NODES, ATOM_DIM), jnp.float32),
    )(x, a0, a1, Wl, bl.reshape(1, ATOM_DIM))


def kernel(x, edge_i, edge_j, edge_f, W1, b1, W2, b2, Wl, bl):
    edge_i = edge_i.astype(jnp.int32)
    edge_j = edge_j.astype(jnp.int32)
    x_j = _sc_gather(x, edge_j)
    m = _tc_edge_mlp(x_j, edge_f, W1[:ATOM_DIM], W1[ATOM_DIM:], b1, W2, b2)
    zeros = jnp.zeros((AGG_ROWS, HIDDEN), jnp.float32)
    agg2 = _sc_scatter_add(m, edge_i, zeros)
    return _tc_final(x, agg2[0, :N_NODES], agg2[1, :N_NODES], Wl, bl)


# speedup vs baseline: 3.7487x; 1.4565x over previous
import functools

import jax
import jax.numpy as jnp
from jax import lax
from jax.experimental import pallas as pl
from jax.experimental.pallas import tpu as pltpu
from jax.experimental.pallas import tpu_sc as plsc

N_NODES = 10000
N_EDGES = 320000
ATOM_DIM = 128
EDGE_DIM = 16
HIDDEN = 128

NC = 2
NS = 16
NW = NC * NS
EPW = N_EDGES // NW
CB = 80
CHUNKS = EPW // CB
KB = 5
OUTER = CHUNKS // KB

SCB = 128
SFULL = 78
STAIL = EPW - SFULL * SCB
SOUTER = SFULL // 2

AGG_ROWS = 10240


def _sc_gather(xw, ej3):
    mesh = plsc.VectorSubcoreMesh(core_axis_name="c", subcore_axis_name="s")

    @functools.partial(
        pl.kernel, mesh=mesh,
        out_type=jax.ShapeDtypeStruct((N_EDGES, ATOM_DIM), jnp.float32),
        scratch_types=[
            pltpu.VMEM((CHUNKS, CB), jnp.int32),
            pltpu.VMEM((KB, CB, ATOM_DIM), jnp.float32),
            pltpu.SemaphoreType.DMA((KB,)),
            pltpu.SemaphoreType.DMA((KB,)),
        ],
    )
    def k(x_hbm, ej_hbm, out_hbm, idx_v, rows_v, gsem, wsem):
        wid = lax.axis_index("s") * NC + lax.axis_index("c")
        base = wid * EPW
        pltpu.sync_copy(ej_hbm.at[wid], idx_v)

        def outer(t, _):
            for b in range(KB):
                j = t * KB + b

                @pl.when(t > 0)
                def _wait_prev_write():
                    pltpu.make_async_copy(
                        rows_v.at[b], out_hbm.at[pl.ds(base + j * CB, CB)],
                        wsem.at[b]).wait()

                pltpu.make_async_copy(
                    x_hbm.at[idx_v.at[j]], rows_v.at[b], gsem.at[b]).start()
            for b in range(KB):
                j = t * KB + b
                pltpu.make_async_copy(
                    x_hbm.at[idx_v.at[j]], rows_v.at[b], gsem.at[b]).wait()
                pltpu.make_async_copy(
                    rows_v.at[b], out_hbm.at[pl.ds(base + j * CB, CB)],
                    wsem.at[b]).start()
            return 0

        lax.fori_loop(0, OUTER, outer, 0)
        for b in range(KB):
            pltpu.make_async_copy(
                rows_v.at[b], out_hbm.at[pl.ds(base + b * CB, CB)],
                wsem.at[b]).wait()

    return k(xw, ej3)


def _sc_scatter_add(m, ei_main, ei_tail, zeros_nh):
    mesh = plsc.VectorSubcoreMesh(core_axis_name="c", subcore_axis_name="s")
    rows_per_sub = AGG_ROWS // NS

    @functools.partial(
        pl.kernel, mesh=mesh,
        out_type=jax.ShapeDtypeStruct((NC, AGG_ROWS, HIDDEN), jnp.float32),
        scratch_types=[
            pltpu.VMEM((SFULL, SCB), jnp.int32),
            pltpu.VMEM((STAIL,), jnp.int32),
            pltpu.VMEM((2, SCB, HIDDEN), jnp.float32),
            pltpu.VMEM((STAIL, HIDDEN), jnp.float32),
            pltpu.VMEM_SHARED((AGG_ROWS, HIDDEN), jnp.float32),
            pltpu.SemaphoreType.DMA((2,)),
            pltpu.SemaphoreType.DMA((2,)),
        ],
    )
    def k(m_hbm, ei_hbm, eit_hbm, z_hbm, out_hbm,
          idx_v, idxt_v, rows_v, rowst_v, agg_sh, msem, ssem):
        cid = lax.axis_index("c")
        sid = lax.axis_index("s")
        wid = sid * NC + cid
        base = wid * EPW
        pltpu.sync_copy(ei_hbm.at[wid], idx_v)
        pltpu.sync_copy(eit_hbm.at[wid], idxt_v)
        pltpu.sync_copy(z_hbm.at[pl.ds(sid * rows_per_sub, rows_per_sub)],
                        agg_sh.at[pl.ds(sid * rows_per_sub, rows_per_sub)])
        plsc.subcore_barrier()

        def outer(t, _):
            for b in range(2):
                j = t * 2 + b

                @pl.when(t > 0)
                def _wait_prev_scatter():
                    pltpu.make_async_copy(
                        rows_v.at[b], agg_sh.at[idx_v.at[j]], ssem.at[b]).wait()

                pltpu.make_async_copy(
                    m_hbm.at[pl.ds(base + j * SCB, SCB)], rows_v.at[b],
                    msem.at[b]).start()
            for b in range(2):
                j = t * 2 + b
                pltpu.make_async_copy(
                    m_hbm.at[pl.ds(base + j * SCB, SCB)], rows_v.at[b],
                    msem.at[b]).wait()
                pltpu.make_async_copy(
                    rows_v.at[b], agg_sh.at[idx_v.at[j]],
                    ssem.at[b]).start(add=True)
            return 0

        lax.fori_loop(0, SOUTER, outer, 0)
        for b in range(2):
            pltpu.make_async_copy(
                rows_v.at[b], agg_sh.at[idx_v.at[b]], ssem.at[b]).wait()
        pltpu.sync_copy(m_hbm.at[pl.ds(base + SFULL * SCB, STAIL)], rowst_v)
        pltpu.sync_copy(rowst_v, agg_sh.at[idxt_v], add=True)
        plsc.subcore_barrier()
        pltpu.sync_copy(agg_sh.at[pl.ds(sid * rows_per_sub, rows_per_sub)],
                        out_hbm.at[cid, pl.ds(sid * rows_per_sub, rows_per_sub)])

    return k(m, ei_main, ei_tail, zeros_nh)


def _tc_edge_mlp(x_j, edge_f, W1a, W1b, b1, W2, b2):
    BE = 2560

    def body(xj_ref, f_ref, w1a, w1b, b1r, w2, b2r, out_ref):
        xjb = xj_ref[...].astype(jnp.bfloat16)
        h = jnp.dot(xjb, w1a[...], preferred_element_type=jnp.float32)
        h = h + jnp.dot(f_ref[...], w1b[...], preferred_element_type=jnp.float32)
        h = h + b1r[...]
        h = h * jax.nn.sigmoid(h)
        hb = h.astype(jnp.bfloat16)
        mm = jnp.dot(hb, w2[...], preferred_element_type=jnp.float32) + b2r[...]
        out_ref[...] = mm * jax.nn.sigmoid(mm)

    return pl.pallas_call(
        body,
        grid=(N_EDGES // BE,),
        in_specs=[
            pl.BlockSpec((BE, ATOM_DIM), lambda i: (i, 0)),
            pl.BlockSpec((BE, EDGE_DIM), lambda i: (i, 0)),
            pl.BlockSpec((ATOM_DIM, HIDDEN), lambda i: (0, 0)),
            pl.BlockSpec((EDGE_DIM, HIDDEN), lambda i: (0, 0)),
            pl.BlockSpec((1, HIDDEN), lambda i: (0, 0)),
            pl.BlockSpec((HIDDEN, HIDDEN), lambda i: (0, 0)),
            pl.BlockSpec((1, HIDDEN), lambda i: (0, 0)),
        ],
        out_specs=pl.BlockSpec((BE, HIDDEN), lambda i: (i, 0)),
        out_shape=jax.ShapeDtypeStruct((N_EDGES, HIDDEN), jnp.float32),
    )(x_j, edge_f, W1a, W1b, b1.reshape(1, HIDDEN), W2, b2.reshape(1, HIDDEN))


def _tc_final(x, a0, a1, Wl, bl):
    BN = 1000

    def body(x_ref, a0_ref, a1_ref, wl, blr, out_ref):
        a = a0_ref[...] + a1_ref[...]
        t = x_ref[...] + jnp.dot(a, wl[...], preferred_element_type=jnp.float32)
        t = t + blr[...]
        out_ref[...] = t * jax.nn.sigmoid(t)

    return pl.pallas_call(
        body,
        grid=(N_NODES // BN,),
        in_specs=[
            pl.BlockSpec((BN, ATOM_DIM), lambda i: (i, 0)),
            pl.BlockSpec((BN, HIDDEN), lambda i: (i, 0)),
            pl.BlockSpec((BN, HIDDEN), lambda i: (i, 0)),
            pl.BlockSpec((HIDDEN, ATOM_DIM), lambda i: (0, 0)),
            pl.BlockSpec((1, ATOM_DIM), lambda i: (0, 0)),
        ],
        out_specs=pl.BlockSpec((BN, ATOM_DIM), lambda i: (i, 0)),
        out_shape=jax.ShapeDtypeStruct((N_NODES, ATOM_DIM), jnp.float32),
    )(x, a0, a1, Wl, bl.reshape(1, ATOM_DIM))


def kernel(x, edge_i, edge_j, edge_f, W1, b1, W2, b2, Wl, bl):
    ei2 = edge_i.astype(jnp.int32).reshape(NW, EPW)
    ei_main = ei2[:, :SFULL * SCB].reshape(NW, SFULL, SCB)
    ei_tail = ei2[:, SFULL * SCB:]
    edge_j = edge_j.astype(jnp.int32).reshape(NW, CHUNKS, CB)
    x_j = _sc_gather(x, edge_j)
    m = _tc_edge_mlp(x_j, edge_f.astype(jnp.bfloat16),
                     W1[:ATOM_DIM].astype(jnp.bfloat16),
                     W1[ATOM_DIM:].astype(jnp.bfloat16),
                     b1, W2.astype(jnp.bfloat16), b2)
    zeros = jnp.zeros((AGG_ROWS, HIDDEN), jnp.float32)
    agg2 = _sc_scatter_add(m, ei_main, ei_tail, zeros)
    return _tc_final(x, agg2[0, :N_NODES], agg2[1, :N_NODES], Wl, bl)
